# baseline (speedup 1.0000x reference)
import functools

import jax
import jax.numpy as jnp
from jax import lax
from jax.experimental import pallas as pl
from jax.experimental.pallas import tpu as pltpu
from jax.experimental.pallas import tpu_sc as plsc

VOCAB = 1000000
EMB = 64
HID = 256
B = 16384
L = 50
NTAB = 3

NW = 32
ROWS = NTAB * B
RPT = B // NW
C = 16
CL = C * L
NCHT = RPT // C
NCH = NTAB * NCHT


def _pool_body(s1_hbm, s2_hbm, w_hbm, emb_hbm, out_hbm,
               idx0, idx1, rows0, rows1, out0, out1,
               gsem0, gsem1, isem0, isem1, osem0, osem1):
    nc = 2
    wid = lax.axis_index("s") * nc + lax.axis_index("c")
    tabs = [s1_hbm, s2_hbm, w_hbm]

    def idx_off(g):
        return (wid * RPT + (g % NCHT) * C) * L

    def out_base(g):
        return (g // NCHT) * B + wid * RPT + (g % NCHT) * C

    def idx_fetch(g, idx_v, isem):
        t = g // NCHT
        off = idx_off(g)
        for ti in range(NTAB):
            @pl.when(t == ti)
            def _():
                pltpu.async_copy(tabs[ti].at[pl.ds(off, CL)], idx_v, isem)

    def reduce_chunk(rows_v, out_v):
        def crow(c, _):
            r0 = c * L
            a = [rows_v[r0, pl.ds(16 * d, 16)] for d in range(4)]
            b = [rows_v[r0 + 1, pl.ds(16 * d, 16)] for d in range(4)]
            for l in range(2, L, 2):
                for d in range(4):
                    a[d] = a[d] + rows_v[r0 + l, pl.ds(16 * d, 16)]
            for l in range(3, L, 2):
                for d in range(4):
                    b[d] = b[d] + rows_v[r0 + l, pl.ds(16 * d, 16)]
            for d in range(4):
                out_v[c, pl.ds(16 * d, 16)] = (a[d] + b[d]) * jnp.float32(1.0 / L)
            return 0

        lax.fori_loop(0, C, crow, 0)

    def phase(g, idx_a, rows_a, gsem_a, isem_a, out_a, osem_a,
              idx_b, rows_b, gsem_b, isem_b):
        @pl.when(g + 1 < NCH)
        def _():
            pltpu.make_async_copy(
                s1_hbm.at[pl.ds(0, CL)], idx_b, isem_b).wait()
            pltpu.async_copy(emb_hbm.at[idx_b], rows_b, gsem_b)

        pltpu.make_async_copy(emb_hbm.at[idx_a], rows_a, gsem_a).wait()

        @pl.when(g + 2 < NCH)
        def _():
            idx_fetch(g + 2, idx_a, isem_a)

        @pl.when(g >= 2)
        def _():
            pltpu.make_async_copy(
                out_a, out_hbm.at[pl.ds(out_base(g - 2), C)], osem_a).wait()

        reduce_chunk(rows_a, out_a)
        pltpu.async_copy(out_a, out_hbm.at[pl.ds(out_base(g), C)], osem_a)

    pltpu.sync_copy(s1_hbm.at[pl.ds(wid * RPT * L, CL)], idx0)
    pltpu.async_copy(emb_hbm.at[idx0], rows0, gsem0)
    pltpu.async_copy(s1_hbm.at[pl.ds((wid * RPT + C) * L, CL)], idx1, isem1)

    def it(i, _):
        phase(2 * i, idx0, rows0, gsem0, isem0, out0, osem0,
              idx1, rows1, gsem1, isem1)
        phase(2 * i + 1, idx1, rows1, gsem1, isem1, out1, osem1,
              idx0, rows0, gsem0, isem0)
        return 0

    lax.fori_loop(0, NCH // 2, it, 0)

    pltpu.make_async_copy(
        out0, out_hbm.at[pl.ds(out_base(NCH - 2), C)], osem0).wait()
    pltpu.make_async_copy(
        out1, out_hbm.at[pl.ds(out_base(NCH - 1), C)], osem1).wait()


_pool = functools.partial(
    pl.kernel,
    mesh=plsc.VectorSubcoreMesh(core_axis_name="c", subcore_axis_name="s"),
    out_type=jax.ShapeDtypeStruct((ROWS, EMB), jnp.float32),
    compiler_params=pltpu.CompilerParams(use_tc_tiling_on_sc=False),
    scratch_types=[
        pltpu.VMEM((CL,), jnp.int32),
        pltpu.VMEM((CL,), jnp.int32),
        pltpu.VMEM((CL, EMB), jnp.float32),
        pltpu.VMEM((CL, EMB), jnp.float32),
        pltpu.VMEM((C, EMB), jnp.float32),
        pltpu.VMEM((C, EMB), jnp.float32),
        pltpu.SemaphoreType.DMA,
        pltpu.SemaphoreType.DMA,
        pltpu.SemaphoreType.DMA,
        pltpu.SemaphoreType.DMA,
        pltpu.SemaphoreType.DMA,
        pltpu.SemaphoreType.DMA,
    ],
)(_pool_body)


VTAIL = (VOCAB // 128) * 128
CT = 7
VB = CT * 128
NBLK = VTAIL // VB
BWORDS = VB * EMB


def _tr_body(embt_hbm, tail_hbm, out_hbm, buf, obuf, tbuf, sem):
    nc = 2
    wid = lax.axis_index("s") * nc + lax.axis_index("c")
    lo = wid * NBLK // NW
    hi = (wid + 1) * NBLK // NW

    iota = lax.iota(jnp.int32, 16)
    rows = [iota + 16 * k for k in range(4)]

    def blk(b, _):
        pltpu.async_copy(embt_hbm.at[:, pl.ds(b * VB, VB)], buf, sem).wait()

        def vcol(i, _):
            v0 = i * 4
            for u in range(4):
                v = v0 + u
                cols = jnp.broadcast_to(v, (16,)).astype(jnp.int32)
                for k in range(4):
                    g = plsc.load_gather(buf, [rows[k], cols])
                    obuf[pl.ds(v * EMB + 16 * k, 16)] = g
            return 0

        lax.fori_loop(0, VB // 4, vcol, 0)
        pltpu.async_copy(obuf, out_hbm.at[pl.ds(b * BWORDS, BWORDS)],
                         sem).wait()
        return 0

    lax.fori_loop(lo, hi, blk, 0)

    @pl.when(wid == NW - 1)
    def _():
        pltpu.async_copy(tail_hbm, tbuf, sem).wait()
        pltpu.async_copy(
            tbuf, out_hbm.at[pl.ds(VTAIL * EMB, (VOCAB - VTAIL) * EMB)],
            sem).wait()


_transpose = functools.partial(
    pl.kernel,
    mesh=plsc.VectorSubcoreMesh(core_axis_name="c", subcore_axis_name="s"),
    out_type=jax.ShapeDtypeStruct((VOCAB * EMB,), jnp.float32),
    compiler_params=pltpu.CompilerParams(use_tc_tiling_on_sc=True,
                                         needs_layout_passes=False),
    scratch_types=[
        pltpu.VMEM((EMB, VB), jnp.float32),
        pltpu.VMEM((BWORDS,), jnp.float32),
        pltpu.VMEM(((VOCAB - VTAIL) * EMB,), jnp.float32),
        pltpu.SemaphoreType.DMA,
    ],
)(_tr_body)


RBLK = 1024


def _mlp_body(p1, p2, p3, xr, w1, w2, w3, wx, b1, w2r, b2, out_ref):
    h = jnp.dot(p1[...], w1[...], preferred_element_type=jnp.float32)
    h = h + jnp.dot(p2[...], w2[...], preferred_element_type=jnp.float32)
    h = h + jnp.dot(p3[...], w3[...], preferred_element_type=jnp.float32)
    h = h + xr[...] * wx[...] + b1[...]
    h = jnp.maximum(h, 0.0)
    z = jnp.sum(h * w2r[...], axis=1, keepdims=True) + b2[...]
    out_ref[...] = 1.0 / (1.0 + jnp.exp(-z))


def _mlp(pooled, x2, w1, w2, w3, wx, b1, w2r, b2):
    grid = B // RBLK
    full = lambda shape: pl.BlockSpec(shape, lambda i: (0, 0))
    return pl.pallas_call(
        _mlp_body,
        grid=(grid,),
        in_specs=[
            pl.BlockSpec((RBLK, EMB), lambda i: (i, 0)),
            pl.BlockSpec((RBLK, EMB), lambda i: (i + B // RBLK, 0)),
            pl.BlockSpec((RBLK, EMB), lambda i: (i + 2 * (B // RBLK), 0)),
            pl.BlockSpec((RBLK, 1), lambda i: (i, 0)),
            full((EMB, HID)),
            full((EMB, HID)),
            full((EMB, HID)),
            full((1, HID)),
            full((1, HID)),
            full((1, HID)),
            full((1, 1)),
        ],
        out_specs=pl.BlockSpec((RBLK, 1), lambda i: (i, 0)),
        out_shape=jax.ShapeDtypeStruct((B, 1), jnp.float32),
    )(pooled, pooled, pooled, x2, w1, w2, w3, wx, b1, w2r, b2)


def kernel(s1, s2, W, x, emb, fc1_w, fc1_b, fc2_w, fc2_b):
    tail = emb[VTAIL:].reshape(-1)
    embl = _transpose(emb.T, tail).reshape(VOCAB, EMB)
    pooled = _pool(s1.reshape(-1), s2.reshape(-1), W.reshape(-1), embl)
    w1 = fc1_w[:, :EMB].T
    w2 = fc1_w[:, EMB:2 * EMB].T
    w3 = fc1_w[:, 2 * EMB:3 * EMB].T
    wx = fc1_w[:, 3 * EMB][None, :]
    b1 = fc1_b[None, :]
    b2 = fc2_b[None, :]
    return _mlp(pooled, x[:, None], w1, w2, w3, wx, b1, fc2_w, b2)


# speedup vs baseline: 1.4132x; 1.4132x over previous
import functools

import jax
import jax.numpy as jnp
from jax import lax
from jax.experimental import pallas as pl
from jax.experimental.pallas import tpu as pltpu
from jax.experimental.pallas import tpu_sc as plsc

VOCAB = 1000000
EMB = 64
HID = 256
B = 16384
L = 50
NTAB = 3

NW = 32
ROWS = NTAB * B
RPT = B // NW
C = 16
CL = C * L
NCHT = RPT // C
NCH = NTAB * NCHT


def _pool_body(s1_hbm, s2_hbm, w_hbm, emb_hbm, out_hbm,
               idx0, idx1, rows0, rows1, out0, out1,
               gsem0, gsem1, isem0, isem1, osem0, osem1):
    nc = 2
    wid = lax.axis_index("s") * nc + lax.axis_index("c")
    tabs = [s1_hbm, s2_hbm, w_hbm]

    def idx_off(g):
        return (wid * RPT + (g % NCHT) * C) * L

    def out_base(g):
        return (g // NCHT) * B + wid * RPT + (g % NCHT) * C

    def idx_fetch(g, idx_v, isem):
        t = g // NCHT
        off = idx_off(g)
        for ti in range(NTAB):
            @pl.when(t == ti)
            def _():
                pltpu.async_copy(tabs[ti].at[pl.ds(off, CL)], idx_v, isem)

    def reduce_chunk(rows_v, out_v):
        def crow(c, _):
            r0 = c * L
            a = [rows_v[r0, pl.ds(16 * d, 16)] for d in range(4)]
            b = [rows_v[r0 + 1, pl.ds(16 * d, 16)] for d in range(4)]
            for l in range(2, L, 2):
                for d in range(4):
                    a[d] = a[d] + rows_v[r0 + l, pl.ds(16 * d, 16)]
            for l in range(3, L, 2):
                for d in range(4):
                    b[d] = b[d] + rows_v[r0 + l, pl.ds(16 * d, 16)]
            for d in range(4):
                out_v[c, pl.ds(16 * d, 16)] = (a[d] + b[d]) * jnp.float32(1.0 / L)
            return 0

        lax.fori_loop(0, C, crow, 0)

    def phase(g, idx_a, rows_a, gsem_a, isem_a, out_a, osem_a,
              idx_b, rows_b, gsem_b, isem_b):
        @pl.when(g + 1 < NCH)
        def _():
            pltpu.make_async_copy(
                s1_hbm.at[pl.ds(0, CL)], idx_b, isem_b).wait()
            pltpu.async_copy(emb_hbm.at[idx_b], rows_b, gsem_b)

        pltpu.make_async_copy(emb_hbm.at[idx_a], rows_a, gsem_a).wait()

        @pl.when(g + 2 < NCH)
        def _():
            idx_fetch(g + 2, idx_a, isem_a)

        @pl.when(g >= 2)
        def _():
            pltpu.make_async_copy(
                out_a, out_hbm.at[pl.ds(out_base(g - 2), C)], osem_a).wait()

        reduce_chunk(rows_a, out_a)
        pltpu.async_copy(out_a, out_hbm.at[pl.ds(out_base(g), C)], osem_a)

    pltpu.sync_copy(s1_hbm.at[pl.ds(wid * RPT * L, CL)], idx0)
    pltpu.async_copy(emb_hbm.at[idx0], rows0, gsem0)
    pltpu.async_copy(s1_hbm.at[pl.ds((wid * RPT + C) * L, CL)], idx1, isem1)

    def it(i, _):
        phase(2 * i, idx0, rows0, gsem0, isem0, out0, osem0,
              idx1, rows1, gsem1, isem1)
        phase(2 * i + 1, idx1, rows1, gsem1, isem1, out1, osem1,
              idx0, rows0, gsem0, isem0)
        return 0

    lax.fori_loop(0, NCH // 2, it, 0)

    pltpu.make_async_copy(
        out0, out_hbm.at[pl.ds(out_base(NCH - 2), C)], osem0).wait()
    pltpu.make_async_copy(
        out1, out_hbm.at[pl.ds(out_base(NCH - 1), C)], osem1).wait()


_pool = functools.partial(
    pl.kernel,
    mesh=plsc.VectorSubcoreMesh(core_axis_name="c", subcore_axis_name="s"),
    out_type=jax.ShapeDtypeStruct((ROWS, EMB), jnp.float32),
    compiler_params=pltpu.CompilerParams(use_tc_tiling_on_sc=False),
    scratch_types=[
        pltpu.VMEM((CL,), jnp.int32),
        pltpu.VMEM((CL,), jnp.int32),
        pltpu.VMEM((CL, EMB), jnp.float32),
        pltpu.VMEM((CL, EMB), jnp.float32),
        pltpu.VMEM((C, EMB), jnp.float32),
        pltpu.VMEM((C, EMB), jnp.float32),
        pltpu.SemaphoreType.DMA,
        pltpu.SemaphoreType.DMA,
        pltpu.SemaphoreType.DMA,
        pltpu.SemaphoreType.DMA,
        pltpu.SemaphoreType.DMA,
        pltpu.SemaphoreType.DMA,
    ],
)(_pool_body)


RB = 512
NFB = VOCAB // RB
TAILR = VOCAB - NFB * RB


def _compact_body(emb_hbm, out_hbm, buf, flat, sem):
    nc = 2
    wid = lax.axis_index("s") * nc + lax.axis_index("c")
    lo = wid * NFB // NW
    hi = (wid + 1) * NFB // NW

    def compact(nrows, r0):
        pltpu.async_copy(emb_hbm.at[pl.ds(r0, nrows), :],
                         buf.at[pl.ds(0, nrows), :], sem).wait()

        def rrow(r, _):
            for d in range(4):
                flat[pl.ds(r * EMB + 16 * d, 16)] = buf[r, pl.ds(16 * d, 16)]
            return 0

        lax.fori_loop(0, nrows, rrow, 0)
        pltpu.async_copy(flat.at[pl.ds(0, nrows * EMB)],
                         out_hbm.at[pl.ds(r0 * EMB, nrows * EMB)], sem).wait()

    def blk(b, _):
        compact(RB, b * RB)
        return 0

    lax.fori_loop(lo, hi, blk, 0)

    @pl.when(wid == NW - 1)
    def _():
        compact(TAILR, NFB * RB)


_compact = functools.partial(
    pl.kernel,
    mesh=plsc.VectorSubcoreMesh(core_axis_name="c", subcore_axis_name="s"),
    out_type=jax.ShapeDtypeStruct((VOCAB * EMB,), jnp.float32),
    compiler_params=pltpu.CompilerParams(use_tc_tiling_on_sc=True,
                                         needs_layout_passes=False),
    scratch_types=[
        pltpu.VMEM((RB, EMB), jnp.float32),
        pltpu.VMEM((RB * EMB,), jnp.float32),
        pltpu.SemaphoreType.DMA,
    ],
)(_compact_body)


RBLK = 1024


def _mlp_body(p1, p2, p3, xr, w1, w2, w3, wx, b1, w2r, b2, out_ref):
    h = jnp.dot(p1[...], w1[...], preferred_element_type=jnp.float32)
    h = h + jnp.dot(p2[...], w2[...], preferred_element_type=jnp.float32)
    h = h + jnp.dot(p3[...], w3[...], preferred_element_type=jnp.float32)
    h = h + xr[...] * wx[...] + b1[...]
    h = jnp.maximum(h, 0.0)
    z = jnp.sum(h * w2r[...], axis=1, keepdims=True) + b2[...]
    out_ref[...] = 1.0 / (1.0 + jnp.exp(-z))


def _mlp(pooled, x2, w1, w2, w3, wx, b1, w2r, b2):
    grid = B // RBLK
    full = lambda shape: pl.BlockSpec(shape, lambda i: (0, 0))
    return pl.pallas_call(
        _mlp_body,
        grid=(grid,),
        in_specs=[
            pl.BlockSpec((RBLK, EMB), lambda i: (i, 0)),
            pl.BlockSpec((RBLK, EMB), lambda i: (i + B // RBLK, 0)),
            pl.BlockSpec((RBLK, EMB), lambda i: (i + 2 * (B // RBLK), 0)),
            pl.BlockSpec((RBLK, 1), lambda i: (i, 0)),
            full((EMB, HID)),
            full((EMB, HID)),
            full((EMB, HID)),
            full((1, HID)),
            full((1, HID)),
            full((1, HID)),
            full((1, 1)),
        ],
        out_specs=pl.BlockSpec((RBLK, 1), lambda i: (i, 0)),
        out_shape=jax.ShapeDtypeStruct((B, 1), jnp.float32),
    )(pooled, pooled, pooled, x2, w1, w2, w3, wx, b1, w2r, b2)


def kernel(s1, s2, W, x, emb, fc1_w, fc1_b, fc2_w, fc2_b):
    embl = _compact(emb).reshape(VOCAB, EMB)
    pooled = _pool(s1.reshape(-1), s2.reshape(-1), W.reshape(-1), embl)
    w1 = fc1_w[:, :EMB].T
    w2 = fc1_w[:, EMB:2 * EMB].T
    w3 = fc1_w[:, 2 * EMB:3 * EMB].T
    wx = fc1_w[:, 3 * EMB][None, :]
    b1 = fc1_b[None, :]
    b2 = fc2_b[None, :]
    return _mlp(pooled, x[:, None], w1, w2, w3, wx, b1, fc2_w, b2)


# speedup vs baseline: 2.1481x; 1.5200x over previous
import functools

import jax
import jax.numpy as jnp
from jax import lax
from jax.experimental import pallas as pl
from jax.experimental.pallas import tpu as pltpu
from jax.experimental.pallas import tpu_sc as plsc

VOCAB = 1000000
EMB = 64
HID = 256
B = 16384
L = 50
NTAB = 3

NW = 32
ROWS = NTAB * B
RPT = B // NW
C = 16
CL = C * L
NCHT = RPT // C
NCH = NTAB * NCHT


def _pool_body(s1_hbm, s2_hbm, w_hbm, emb_hbm, out_hbm,
               idx0, idx1, rows0, rows1, out0, out1,
               gsem0, gsem1, isem0, isem1, osem0, osem1):
    nc = 2
    wid = lax.axis_index("s") * nc + lax.axis_index("c")
    tabs = [s1_hbm, s2_hbm, w_hbm]

    def idx_off(g):
        return (wid * RPT + (g % NCHT) * C) * L

    def out_base(g):
        return (g // NCHT) * B + wid * RPT + (g % NCHT) * C

    def idx_fetch(g, idx_v, isem):
        t = g // NCHT
        off = idx_off(g)
        for ti in range(NTAB):
            @pl.when(t == ti)
            def _():
                pltpu.async_copy(tabs[ti].at[pl.ds(off, CL)], idx_v, isem)

    def reduce_chunk(rows_v, out_v):
        def crow(c, _):
            r0 = c * L
            a = [rows_v[r0, pl.ds(16 * d, 16)] for d in range(4)]
            b = [rows_v[r0 + 1, pl.ds(16 * d, 16)] for d in range(4)]
            for l in range(2, L, 2):
                for d in range(4):
                    a[d] = a[d] + rows_v[r0 + l, pl.ds(16 * d, 16)]
            for l in range(3, L, 2):
                for d in range(4):
                    b[d] = b[d] + rows_v[r0 + l, pl.ds(16 * d, 16)]
            for d in range(4):
                out_v[c, pl.ds(16 * d, 16)] = (a[d] + b[d]) * jnp.float32(1.0 / L)
            return 0

        lax.fori_loop(0, C, crow, 0)

    def phase(g, idx_a, rows_a, gsem_a, isem_a, out_a, osem_a,
              idx_b, rows_b, gsem_b, isem_b):
        @pl.when(g + 1 < NCH)
        def _():
            pltpu.make_async_copy(
                s1_hbm.at[pl.ds(0, CL)], idx_b, isem_b).wait()
            pltpu.async_copy(emb_hbm.at[idx_b], rows_b, gsem_b)

        pltpu.make_async_copy(emb_hbm.at[idx_a], rows_a, gsem_a).wait()

        @pl.when(g + 2 < NCH)
        def _():
            idx_fetch(g + 2, idx_a, isem_a)

        @pl.when(g >= 2)
        def _():
            pltpu.make_async_copy(
                out_a, out_hbm.at[pl.ds(out_base(g - 2), C)], osem_a).wait()

        reduce_chunk(rows_a, out_a)
        pltpu.async_copy(out_a, out_hbm.at[pl.ds(out_base(g), C)], osem_a)

    pltpu.sync_copy(s1_hbm.at[pl.ds(wid * RPT * L, CL)], idx0)
    pltpu.async_copy(emb_hbm.at[idx0], rows0, gsem0)
    pltpu.async_copy(s1_hbm.at[pl.ds((wid * RPT + C) * L, CL)], idx1, isem1)

    def it(i, _):
        phase(2 * i, idx0, rows0, gsem0, isem0, out0, osem0,
              idx1, rows1, gsem1, isem1)
        phase(2 * i + 1, idx1, rows1, gsem1, isem1, out1, osem1,
              idx0, rows0, gsem0, isem0)
        return 0

    lax.fori_loop(0, NCH // 2, it, 0)

    pltpu.make_async_copy(
        out0, out_hbm.at[pl.ds(out_base(NCH - 2), C)], osem0).wait()
    pltpu.make_async_copy(
        out1, out_hbm.at[pl.ds(out_base(NCH - 1), C)], osem1).wait()


_pool = functools.partial(
    pl.kernel,
    mesh=plsc.VectorSubcoreMesh(core_axis_name="c", subcore_axis_name="s"),
    out_type=jax.ShapeDtypeStruct((ROWS, EMB), jnp.float32),
    compiler_params=pltpu.CompilerParams(use_tc_tiling_on_sc=False),
    scratch_types=[
        pltpu.VMEM((CL,), jnp.int32),
        pltpu.VMEM((CL,), jnp.int32),
        pltpu.VMEM((CL, EMB), jnp.float32),
        pltpu.VMEM((CL, EMB), jnp.float32),
        pltpu.VMEM((C, EMB), jnp.float32),
        pltpu.VMEM((C, EMB), jnp.float32),
        pltpu.SemaphoreType.DMA,
        pltpu.SemaphoreType.DMA,
        pltpu.SemaphoreType.DMA,
        pltpu.SemaphoreType.DMA,
        pltpu.SemaphoreType.DMA,
        pltpu.SemaphoreType.DMA,
    ],
)(_pool_body)


RBLK = 1024


def _mlp_body(p1, p2, p3, xr, w1, w2, w3, wx, b1, w2r, b2, out_ref):
    h = jnp.dot(p1[...], w1[...], preferred_element_type=jnp.float32)
    h = h + jnp.dot(p2[...], w2[...], preferred_element_type=jnp.float32)
    h = h + jnp.dot(p3[...], w3[...], preferred_element_type=jnp.float32)
    h = h + xr[...] * wx[...] + b1[...]
    h = jnp.maximum(h, 0.0)
    z = jnp.sum(h * w2r[...], axis=1, keepdims=True) + b2[...]
    out_ref[...] = 1.0 / (1.0 + jnp.exp(-z))


def _mlp(pooled, x2, w1, w2, w3, wx, b1, w2r, b2):
    grid = B // RBLK
    full = lambda shape: pl.BlockSpec(shape, lambda i: (0, 0))
    return pl.pallas_call(
        _mlp_body,
        grid=(grid,),
        in_specs=[
            pl.BlockSpec((RBLK, EMB), lambda i: (i, 0)),
            pl.BlockSpec((RBLK, EMB), lambda i: (i + B // RBLK, 0)),
            pl.BlockSpec((RBLK, EMB), lambda i: (i + 2 * (B // RBLK), 0)),
            pl.BlockSpec((RBLK, 1), lambda i: (i, 0)),
            full((EMB, HID)),
            full((EMB, HID)),
            full((EMB, HID)),
            full((1, HID)),
            full((1, HID)),
            full((1, HID)),
            full((1, 1)),
        ],
        out_specs=pl.BlockSpec((RBLK, 1), lambda i: (i, 0)),
        out_shape=jax.ShapeDtypeStruct((B, 1), jnp.float32),
    )(pooled, pooled, pooled, x2, w1, w2, w3, wx, b1, w2r, b2)


def kernel(s1, s2, W, x, emb, fc1_w, fc1_b, fc2_w, fc2_b):
    pooled = _pool(s1.reshape(-1), s2.reshape(-1), W.reshape(-1), emb)
    w1 = fc1_w[:, :EMB].T
    w2 = fc1_w[:, EMB:2 * EMB].T
    w3 = fc1_w[:, 2 * EMB:3 * EMB].T
    wx = fc1_w[:, 3 * EMB][None, :]
    b1 = fc1_b[None, :]
    b2 = fc2_b[None, :]
    return _mlp(pooled, x[:, None], w1, w2, w3, wx, b1, fc2_w, b2)


# speedup vs baseline: 2.1589x; 1.0051x over previous
import functools

import jax
import jax.numpy as jnp
from jax import lax
from jax.experimental import pallas as pl
from jax.experimental.pallas import tpu as pltpu
from jax.experimental.pallas import tpu_sc as plsc

VOCAB = 1000000
EMB = 64
HID = 256
B = 16384
L = 50
NTAB = 3

NW = 32
ROWS = NTAB * B
RPT = B // NW
C = 8
CL = C * L
NCHT = RPT // C
NCH = NTAB * NCHT


def _pool_body(s1_hbm, s2_hbm, w_hbm, emb_hbm, out_hbm,
               idx0, idx1, idx2, rows0, rows1, rows2, out0, out1, out2,
               gsem0, gsem1, gsem2, isem0, isem1, isem2,
               osem0, osem1, osem2):
    nc = 2
    wid = lax.axis_index("s") * nc + lax.axis_index("c")
    tabs = [s1_hbm, s2_hbm, w_hbm]

    def idx_off(g):
        return (wid * RPT + (g % NCHT) * C) * L

    def out_base(g):
        return (g // NCHT) * B + wid * RPT + (g % NCHT) * C

    def idx_fetch(g, idx_v, isem):
        t = g // NCHT
        off = idx_off(g)
        for ti in range(NTAB):
            @pl.when(t == ti)
            def _():
                pltpu.async_copy(tabs[ti].at[pl.ds(off, CL)], idx_v, isem)

    def reduce_chunk(rows_v, out_v):
        def crow(c, _):
            r0 = c * L
            a = [rows_v[r0, pl.ds(16 * d, 16)] for d in range(4)]
            b = [rows_v[r0 + 1, pl.ds(16 * d, 16)] for d in range(4)]
            for l in range(2, L, 2):
                for d in range(4):
                    a[d] = a[d] + rows_v[r0 + l, pl.ds(16 * d, 16)]
            for l in range(3, L, 2):
                for d in range(4):
                    b[d] = b[d] + rows_v[r0 + l, pl.ds(16 * d, 16)]
            for d in range(4):
                out_v[c, pl.ds(16 * d, 16)] = (a[d] + b[d]) * jnp.float32(1.0 / L)
            return 0

        lax.fori_loop(0, C, crow, 0)

    def phase(g, idx_a, rows_a, gsem_a, isem_a, out_a, osem_a,
              idx_c, rows_c, gsem_c, isem_c):
        pltpu.make_async_copy(emb_hbm.at[idx_a], rows_a, gsem_a).wait()

        @pl.when(g + 3 < NCH)
        def _():
            idx_fetch(g + 3, idx_a, isem_a)

        @pl.when(g + 2 < NCH)
        def _():
            pltpu.make_async_copy(
                s1_hbm.at[pl.ds(0, CL)], idx_c, isem_c).wait()
            pltpu.async_copy(emb_hbm.at[idx_c], rows_c, gsem_c)

        @pl.when(g >= 3)
        def _():
            pltpu.make_async_copy(
                out_a, out_hbm.at[pl.ds(out_base(g - 3), C)], osem_a).wait()

        reduce_chunk(rows_a, out_a)
        pltpu.async_copy(out_a, out_hbm.at[pl.ds(out_base(g), C)], osem_a)

    pltpu.sync_copy(s1_hbm.at[pl.ds(wid * RPT * L, CL)], idx0)
    pltpu.async_copy(emb_hbm.at[idx0], rows0, gsem0)
    pltpu.sync_copy(s1_hbm.at[pl.ds((wid * RPT + C) * L, CL)], idx1)
    pltpu.async_copy(emb_hbm.at[idx1], rows1, gsem1)
    pltpu.async_copy(s1_hbm.at[pl.ds((wid * RPT + 2 * C) * L, CL)], idx2,
                     isem2)

    def it(i, _):
        phase(3 * i, idx0, rows0, gsem0, isem0, out0, osem0,
              idx2, rows2, gsem2, isem2)
        phase(3 * i + 1, idx1, rows1, gsem1, isem1, out1, osem1,
              idx0, rows0, gsem0, isem0)
        phase(3 * i + 2, idx2, rows2, gsem2, isem2, out2, osem2,
              idx1, rows1, gsem1, isem1)
        return 0

    lax.fori_loop(0, NCH // 3, it, 0)

    pltpu.make_async_copy(
        out0, out_hbm.at[pl.ds(out_base(NCH - 3), C)], osem0).wait()
    pltpu.make_async_copy(
        out1, out_hbm.at[pl.ds(out_base(NCH - 2), C)], osem1).wait()
    pltpu.make_async_copy(
        out2, out_hbm.at[pl.ds(out_base(NCH - 1), C)], osem2).wait()


_pool = functools.partial(
    pl.kernel,
    mesh=plsc.VectorSubcoreMesh(core_axis_name="c", subcore_axis_name="s"),
    out_type=jax.ShapeDtypeStruct((ROWS, EMB), jnp.float32),
    compiler_params=pltpu.CompilerParams(use_tc_tiling_on_sc=False),
    scratch_types=(
        [pltpu.VMEM((CL,), jnp.int32)] * 3
        + [pltpu.VMEM((CL, EMB), jnp.float32)] * 3
        + [pltpu.VMEM((C, EMB), jnp.float32)] * 3
        + [pltpu.SemaphoreType.DMA] * 9
    ),
)(_pool_body)


RBLK = 1024


def _mlp_body(p1, p2, p3, xr, w1, w2, w3, wx, b1, w2r, b2, out_ref):
    h = jnp.dot(p1[...], w1[...], preferred_element_type=jnp.float32)
    h = h + jnp.dot(p2[...], w2[...], preferred_element_type=jnp.float32)
    h = h + jnp.dot(p3[...], w3[...], preferred_element_type=jnp.float32)
    h = h + xr[...] * wx[...] + b1[...]
    h = jnp.maximum(h, 0.0)
    z = jnp.sum(h * w2r[...], axis=1, keepdims=True) + b2[...]
    out_ref[...] = 1.0 / (1.0 + jnp.exp(-z))


def _mlp(pooled, x2, w1, w2, w3, wx, b1, w2r, b2):
    grid = B // RBLK
    full = lambda shape: pl.BlockSpec(shape, lambda i: (0, 0))
    return pl.pallas_call(
        _mlp_body,
        grid=(grid,),
        in_specs=[
            pl.BlockSpec((RBLK, EMB), lambda i: (i, 0)),
            pl.BlockSpec((RBLK, EMB), lambda i: (i + B // RBLK, 0)),
            pl.BlockSpec((RBLK, EMB), lambda i: (i + 2 * (B // RBLK), 0)),
            pl.BlockSpec((RBLK, 1), lambda i: (i, 0)),
            full((EMB, HID)),
            full((EMB, HID)),
            full((EMB, HID)),
            full((1, HID)),
            full((1, HID)),
            full((1, HID)),
            full((1, 1)),
        ],
        out_specs=pl.BlockSpec((RBLK, 1), lambda i: (i, 0)),
        out_shape=jax.ShapeDtypeStruct((B, 1), jnp.float32),
    )(pooled, pooled, pooled, x2, w1, w2, w3, wx, b1, w2r, b2)


def kernel(s1, s2, W, x, emb, fc1_w, fc1_b, fc2_w, fc2_b):
    pooled = _pool(s1.reshape(-1), s2.reshape(-1), W.reshape(-1), emb)
    w1 = fc1_w[:, :EMB].T
    w2 = fc1_w[:, EMB:2 * EMB].T
    w3 = fc1_w[:, 2 * EMB:3 * EMB].T
    wx = fc1_w[:, 3 * EMB][None, :]
    b1 = fc1_b[None, :]
    b2 = fc2_b[None, :]
    return _mlp(pooled, x[:, None], w1, w2, w3, wx, b1, fc2_w, b2)
